# baseline (speedup 1.0000x reference)
import functools

import jax
import jax.numpy as jnp
from jax import lax
from jax.experimental import pallas as pl
from jax.experimental.pallas import tpu as pltpu
from jax.experimental.pallas import tpu_sc as plsc

NUM_CORES = 2
NUM_SUBCORES = 16
NUM_WORKERS = NUM_CORES * NUM_SUBCORES
LANES = 16
NBUF = 4


def _positional(seq, emb_sz, n=10000):
    positions = jnp.arange(seq)[:, None].astype(jnp.float32)
    pairs = jnp.arange(emb_sz) // 2
    is_even = (jnp.arange(emb_sz) % 2 == 0)
    angle = positions / (n ** (2 * pairs / emb_sz))
    return jnp.where(is_even, jnp.sin(angle), jnp.cos(angle))


def _make_sc_kernel(vocab, emb, n_rows, chunk, subs):
    rows_per_w = n_rows // NUM_WORKERS
    n_chunks = rows_per_w // chunk
    half = chunk // 2
    wide = 2 * emb
    assert n_chunks % NBUF == 0 and n_chunks >= 2 * NBUF
    assert sum(subs) == chunk and all(s % 8 == 0 for s in subs[:-1])
    mesh = plsc.VectorSubcoreMesh(
        core_axis_name="c", subcore_axis_name="s",
        num_cores=NUM_CORES, num_subcores=NUM_SUBCORES)

    @functools.partial(
        pl.kernel,
        out_type=jax.ShapeDtypeStruct((n_rows // 2, wide), jnp.float32),
        mesh=mesh,
        scratch_types=[
            pltpu.VMEM((rows_per_w,), jnp.int32),
            [pltpu.VMEM((chunk, emb), jnp.float32) for _ in range(NBUF)],
            [pltpu.VMEM((half, wide), jnp.float32) for _ in range(NBUF)],
            pltpu.VMEM((half, wide), jnp.float32),
            [pltpu.SemaphoreType.DMA for _ in range(NBUF)],
            [pltpu.SemaphoreType.DMA for _ in range(NBUF)],
        ],
        compiler_params=pltpu.CompilerParams(use_tc_tiling_on_sc=False),
    )
    def sc_kernel(table_hbm, ids_hbm, pos_hbm, out_hbm,
                  idx_v, gbufs, wbufs, pos_v, gsems, wsems):
        wid = lax.axis_index("s") * NUM_CORES + lax.axis_index("c")
        base = pl.multiple_of(wid * rows_per_w, rows_per_w)
        wbase = pl.multiple_of(wid * (rows_per_w // 2), rows_per_w // 2)

        pltpu.sync_copy(ids_hbm.at[pl.ds(base, rows_per_w)], idx_v)
        pltpu.sync_copy(pos_hbm, pos_v)

        def fire_gather(k, p):
            row0 = pl.multiple_of(k * chunk, chunk)
            off = 0
            for s in subs:
                pltpu.async_copy(
                    table_hbm.at[idx_v.at[pl.ds(row0 + off, s)]],
                    gbufs[p].at[pl.ds(off, s)], gsems[p])
                off += s

        def drain_gather(p):
            pltpu.make_async_copy(
                table_hbm.at[pl.ds(0, chunk)], gbufs[p], gsems[p]).wait()

        def add_pos(p):
            @pl.loop(0, half, unroll=4)
            def _add(t):
                for h in range(2):
                    for k in range(emb // LANES):
                        col = h * emb + k * LANES
                        wbufs[p][t, pl.ds(col, LANES)] = (
                            gbufs[p][2 * t + h, pl.ds(k * LANES, LANES)]
                            + pos_v[t, pl.ds(col, LANES)])

        def fire_write(k, p):
            row0 = pl.multiple_of(k * half, half)
            pltpu.async_copy(wbufs[p], out_hbm.at[pl.ds(wbase + row0, half)],
                             wsems[p])

        def drain_write(p):
            pltpu.make_async_copy(
                wbufs[p], out_hbm.at[pl.ds(wbase, half)], wsems[p]).wait()

        for p in range(NBUF - 1):
            fire_gather(p, p)

        @pl.loop(0, n_chunks, step=NBUF)
        def _pipe(c):
            for p in range(NBUF):
                k = c + p

                @pl.when(k >= 1)
                def _():
                    drain_write((p + NBUF - 1) % NBUF)

                @pl.when(k + NBUF - 1 < n_chunks)
                def _():
                    fire_gather(k + NBUF - 1, (p + NBUF - 1) % NBUF)

                drain_gather(p)
                add_pos(p)
                fire_write(k, p)

        drain_write((n_chunks - 1) % NBUF)

    return sc_kernel


def kernel(input_ids, token_emb_table):
    bs, seq = input_ids.shape
    vocab, emb = token_emb_table.shape
    n_rows = bs * seq

    chunk = 4 * seq
    subs = (128, 72)

    ids_flat = input_ids.reshape(-1).astype(jnp.int32)
    pos_tiled = (jnp.tile(_positional(seq, emb), (chunk // seq, 1))
                 .reshape(chunk // 2, 2 * emb).astype(jnp.float32))

    sc = _make_sc_kernel(vocab, emb, n_rows, chunk, subs)
    out2 = sc(token_emb_table, ids_flat, pos_tiled)
    return out2.reshape(bs, seq, emb)


# speedup vs baseline: 1.3918x; 1.3918x over previous
import functools

import jax
import jax.numpy as jnp
from jax import lax
from jax.experimental import pallas as pl
from jax.experimental.pallas import tpu as pltpu
from jax.experimental.pallas import tpu_sc as plsc

NUM_CORES = 2
NUM_SUBCORES = 16
NUM_WORKERS = NUM_CORES * NUM_SUBCORES
LANES = 16
NBUF = 4


def _positional(seq, emb_sz, n=10000):
    positions = jnp.arange(seq)[:, None].astype(jnp.float32)
    pairs = jnp.arange(emb_sz) // 2
    is_even = (jnp.arange(emb_sz) % 2 == 0)
    angle = positions / (n ** (2 * pairs / emb_sz))
    return jnp.where(is_even, jnp.sin(angle), jnp.cos(angle))


def _make_sc_kernel(vocab, emb, n_rows, chunk, subs):
    rows_per_w = n_rows // NUM_WORKERS
    n_chunks = rows_per_w // chunk
    assert n_chunks % NBUF == 0 and n_chunks >= 2 * NBUF
    assert sum(subs) == chunk and all(s % 8 == 0 for s in subs[:-1])
    mesh = plsc.VectorSubcoreMesh(
        core_axis_name="c", subcore_axis_name="s",
        num_cores=NUM_CORES, num_subcores=NUM_SUBCORES)

    @functools.partial(
        pl.kernel,
        out_type=jax.ShapeDtypeStruct((n_rows, emb), jnp.float32),
        mesh=mesh,
        scratch_types=[
            pltpu.VMEM((rows_per_w,), jnp.int32),
            [pltpu.VMEM((chunk, emb), jnp.float32) for _ in range(NBUF)],
            pltpu.VMEM((chunk, emb), jnp.float32),
            [pltpu.SemaphoreType.DMA for _ in range(NBUF)],
            [pltpu.SemaphoreType.DMA for _ in range(NBUF)],
        ],
        compiler_params=pltpu.CompilerParams(use_tc_tiling_on_sc=False),
    )
    def sc_kernel(table_hbm, ids_hbm, pos_hbm, out_hbm,
                  idx_v, bufs, pos_v, gsems, wsems):
        wid = lax.axis_index("s") * NUM_CORES + lax.axis_index("c")
        base = pl.multiple_of(wid * rows_per_w, rows_per_w)

        pltpu.sync_copy(ids_hbm.at[pl.ds(base, rows_per_w)], idx_v)
        pltpu.sync_copy(pos_hbm, pos_v)

        def fire_gather(k, p):
            row0 = pl.multiple_of(k * chunk, chunk)
            off = 0
            for s in subs:
                pltpu.async_copy(
                    table_hbm.at[idx_v.at[pl.ds(row0 + off, s)]],
                    bufs[p].at[pl.ds(off, s)], gsems[p])
                off += s

        def drain_gather(p):
            pltpu.make_async_copy(
                table_hbm.at[pl.ds(0, chunk)], bufs[p], gsems[p]).wait()

        def add_pos(p):
            @pl.loop(0, chunk, unroll=4)
            def _add(r):
                for k in range(emb // LANES):
                    plsc.addupdate(bufs[p].at[r, pl.ds(k * LANES, LANES)],
                                   pos_v[r, pl.ds(k * LANES, LANES)])

        def fire_write(k, p):
            row0 = pl.multiple_of(k * chunk, chunk)
            pltpu.async_copy(bufs[p], out_hbm.at[pl.ds(base + row0, chunk)],
                             wsems[p])

        def drain_write(p):
            pltpu.make_async_copy(
                bufs[p], out_hbm.at[pl.ds(base, chunk)], wsems[p]).wait()

        for p in range(NBUF - 1):
            fire_gather(p, p)

        @pl.loop(0, n_chunks, step=NBUF)
        def _pipe(c):
            for p in range(NBUF):
                k = c + p

                @pl.when(k >= 1)
                def _():
                    drain_write((p + NBUF - 1) % NBUF)

                @pl.when(k + NBUF - 1 < n_chunks)
                def _():
                    fire_gather(k + NBUF - 1, (p + NBUF - 1) % NBUF)

                drain_gather(p)
                add_pos(p)
                fire_write(k, p)

        drain_write((n_chunks - 1) % NBUF)

    return sc_kernel


def kernel(input_ids, token_emb_table):
    bs, seq = input_ids.shape
    vocab, emb = token_emb_table.shape
    n_rows = bs * seq

    chunk = 4 * seq
    subs = (128, 72)

    ids_flat = input_ids.reshape(-1).astype(jnp.int32)
    pos_tiled = jnp.tile(_positional(seq, emb), (chunk // seq, 1)).astype(jnp.float32)

    sc = _make_sc_kernel(vocab, emb, n_rows, chunk, subs)
    out = sc(token_emb_table, ids_flat, pos_tiled)
    return out.reshape(bs, seq, emb)
